# baseline (speedup 1.0000x reference)
import jax
import jax.numpy as jnp
from jax import lax
from jax.experimental import pallas as pl
from jax.experimental.pallas import tpu as pltpu
from jax.experimental.pallas import tpu_sc as plsc

N = 10000
D = 128
E = 320000
EPS = 1e-5

NC = 2
NS = 16
DH = D // NC
ESUB = E // NS
CH = 80
NCHUNK = ESUB // CH
NPAIR = NCHUNK // 2
HALF = NCHUNK // 2
RU = 80
NUNIT = N // RU
DEGW = 16
SHIFT = 14


def _sc_body(feat_hbm, packed_hbm, out_sum, out_deg,
             packed_v, src_c, dst_c, rows_v, ones_v, zdeg_v,
             table_sh, acc_sh, deg_sh, sem0, sem1):
    cid = lax.axis_index("c")
    sid = lax.axis_index("s")

    pltpu.sync_copy(packed_hbm.at[sid], packed_v)

    zeros16 = jnp.zeros((16,), jnp.float32)
    ones16 = jnp.ones((16,), jnp.float32)

    def fill(r, carry):
        for q in range(DH // 16):
            rows_v[0, r, pl.ds(q * 16, 16)] = zeros16
        ones_v[r] = ones16
        zdeg_v[r] = zeros16
        return carry
    lax.fori_loop(0, RU, fill, 0)

    def init_unit(k, carry):
        u = sid + NS * k

        @pl.when(u < NUNIT)
        def _():
            pltpu.sync_copy(rows_v.at[0], acc_sh.at[pl.ds(u * RU, RU)])
            pltpu.sync_copy(zdeg_v, deg_sh.at[pl.ds(u * RU, RU)])
            pltpu.sync_copy(feat_hbm.at[cid, pl.ds(u * RU, RU)],
                            table_sh.at[pl.ds(u * RU, RU)])
        return carry
    lax.fori_loop(0, (NUNIT + NS - 1) // NS, init_unit, 0)

    plsc.subcore_barrier()

    mask = jnp.full((16,), (1 << SHIFT) - 1, jnp.int32)

    def unpack(j, slot):
        for q in range(CH // 16):
            p = packed_v[j, pl.ds(q * 16, 16)]
            src_c[slot, pl.ds(q * 16, 16)] = p & mask
            dst_c[slot, pl.ds(q * 16, 16)] = jax.lax.shift_right_logical(p, SHIFT)

    def gather(slot, sem):
        return pltpu.make_async_copy(table_sh.at[src_c.at[slot]],
                                     rows_v.at[slot], sem)

    def scatter(j, slot):
        pltpu.sync_copy(rows_v.at[slot], acc_sh.at[dst_c.at[slot]], add=True)
        do_deg = jnp.where(cid == 0, j < HALF, j >= HALF)

        @pl.when(do_deg)
        def _():
            pltpu.sync_copy(ones_v, deg_sh.at[dst_c.at[slot]], add=True)

    unpack(0, 0)
    gather(0, sem0).start()

    def edge_pair(k, carry):
        j0 = 2 * k
        j1 = j0 + 1
        unpack(j1, 1)
        gather(0, sem0).wait()
        gather(1, sem1).start()
        scatter(j0, 0)
        gather(1, sem1).wait()

        @pl.when(k + 1 < NPAIR)
        def _():
            unpack(j0 + 2, 0)
            gather(0, sem0).start()
        scatter(j1, 1)
        return carry
    lax.fori_loop(0, NPAIR, edge_pair, 0)

    plsc.subcore_barrier()

    def writeout(k, carry):
        u = sid + NS * k

        @pl.when(u < NUNIT)
        def _():
            pltpu.sync_copy(acc_sh.at[pl.ds(u * RU, RU)],
                            out_sum.at[cid, pl.ds(u * RU, RU)])
            pltpu.sync_copy(deg_sh.at[pl.ds(u * RU, RU)],
                            out_deg.at[cid, pl.ds(u * RU, RU)])
        return carry
    lax.fori_loop(0, (NUNIT + NS - 1) // NS, writeout, 0)


_sc_segsum = pl.kernel(
    _sc_body,
    out_type=[jax.ShapeDtypeStruct((NC, N, DH), jnp.float32),
              jax.ShapeDtypeStruct((NC, N, DEGW), jnp.float32)],
    mesh=plsc.VectorSubcoreMesh(core_axis_name="c", subcore_axis_name="s"),
    compiler_params=pltpu.CompilerParams(use_tc_tiling_on_sc=False),
    scratch_types=[
        pltpu.VMEM((NCHUNK, CH), jnp.int32),
        pltpu.VMEM((2, CH), jnp.int32),
        pltpu.VMEM((2, CH), jnp.int32),
        pltpu.VMEM((2, CH, DH), jnp.float32),
        pltpu.VMEM((CH, DEGW), jnp.float32),
        pltpu.VMEM((RU, DEGW), jnp.float32),
        pltpu.VMEM_SHARED((N, DH), jnp.float32),
        pltpu.VMEM_SHARED((N, DH), jnp.float32),
        pltpu.VMEM_SHARED((N, DEGW), jnp.float32),
        pltpu.SemaphoreType.DMA,
        pltpu.SemaphoreType.DMA,
    ],
)


def _tc_body(ps_ref, pd_ref, feat_ref, w_ref, b_ref, g_ref, be_ref, out_ref):
    summed = jnp.concatenate([ps_ref[0], ps_ref[1]], axis=1)
    deg = (pd_ref[0] + pd_ref[1])[:, 0:1]
    h = summed / jnp.maximum(deg, 1.0)
    z = lax.dot_general(h, w_ref[...],
                        dimension_numbers=(((1,), (1,)), ((), ())),
                        preferred_element_type=jnp.float32)
    z = z + b_ref[...]
    mean = jnp.mean(z, axis=0, keepdims=True)
    c = z - mean
    var = jnp.mean(c * c, axis=0, keepdims=True)
    zn = c / jnp.sqrt(var + EPS) * g_ref[...] + be_ref[...]
    out_ref[...] = feat_ref[...] + jnp.maximum(zn, 0.0)


def kernel(feature, edge_index, W, b, gamma, beta):
    feat_halves = jnp.stack([feature[:, :DH], feature[:, DH:]])
    packed = (edge_index[0] | (edge_index[1] << SHIFT)).reshape(NS, NCHUNK, CH)
    ps, pd = _sc_segsum(feat_halves, packed)
    return pl.pallas_call(
        _tc_body,
        out_shape=jax.ShapeDtypeStruct((N, D), jnp.float32),
    )(ps, pd, feature, W, b.reshape(1, D), gamma.reshape(1, D),
      beta.reshape(1, D))


# speedup vs baseline: 1.0731x; 1.0731x over previous
import jax
import jax.numpy as jnp
from jax import lax
from jax.experimental import pallas as pl
from jax.experimental.pallas import tpu as pltpu
from jax.experimental.pallas import tpu_sc as plsc

N = 10000
D = 128
E = 320000
EPS = 1e-5

NC = 2
NS = 16
DH = D // NC
ESUB = E // NS
CH = 80
NCHUNK = ESUB // CH
NPAIR = NCHUNK // 2
HALF = NCHUNK // 2
RU = 80
NUNIT = N // RU
DEGW = 16
SHIFT = 14


def _sc_body(feat_hbm, packed_hbm, out_sum, out_deg,
             packed_v, src_c, dst_c, rows_v, ones_v, zdeg_v,
             table_sh, acc_sh, deg_sh, gsem, ssem, dsem):
    cid = lax.axis_index("c")
    sid = lax.axis_index("s")

    pltpu.sync_copy(packed_hbm.at[sid, 0], packed_v)

    zeros16 = jnp.zeros((16,), jnp.float32)
    ones16 = jnp.ones((16,), jnp.float32)

    def fill(r, carry):
        for q in range(DH // 16):
            rows_v[0, r, pl.ds(q * 16, 16)] = zeros16
        ones_v[r] = ones16
        zdeg_v[r] = zeros16
        return carry
    lax.fori_loop(0, RU, fill, 0)

    def init_unit(k, carry):
        u = sid + NS * k

        @pl.when(u < NUNIT)
        def _():
            pltpu.sync_copy(rows_v.at[0], acc_sh.at[pl.ds(u * RU, RU)])
            pltpu.sync_copy(zdeg_v, deg_sh.at[pl.ds(u * RU, RU)])
            pltpu.sync_copy(feat_hbm.at[cid, pl.ds(u * RU, RU)],
                            table_sh.at[pl.ds(u * RU, RU)])
        return carry
    lax.fori_loop(0, (NUNIT + NS - 1) // NS, init_unit, 0)

    plsc.subcore_barrier()

    mask = jnp.full((16,), (1 << SHIFT) - 1, jnp.int32)

    def unpack(j, slot):
        for q in range(CH // 16):
            p = packed_v[j, pl.ds(q * 16, 16)]
            src_c[slot, pl.ds(q * 16, 16)] = p & mask
            dst_c[slot, pl.ds(q * 16, 16)] = jax.lax.shift_right_logical(p, SHIFT)

    def gather(slot):
        return pltpu.make_async_copy(table_sh.at[src_c.at[slot]],
                                     rows_v.at[slot], gsem.at[slot])

    def do_deg(j):
        return jnp.where(cid == 0, j < HALF, j >= HALF)

    def acc_desc(slot):
        return pltpu.make_async_copy(rows_v.at[slot],
                                     acc_sh.at[dst_c.at[slot]], ssem.at[slot])

    def deg_desc(slot):
        return pltpu.make_async_copy(ones_v, deg_sh.at[dst_c.at[slot]],
                                     dsem.at[slot])

    def scatter_start(j, slot):
        pltpu.async_copy(rows_v.at[slot], acc_sh.at[dst_c.at[slot]],
                         ssem.at[slot], add=True)

        @pl.when(do_deg(j))
        def _():
            pltpu.async_copy(ones_v, deg_sh.at[dst_c.at[slot]],
                             dsem.at[slot], add=True)

    def scatter_wait(j, slot):
        acc_desc(slot).wait()

        @pl.when(do_deg(j))
        def _():
            deg_desc(slot).wait()

    def step(j, carry):
        s = j & 3

        @pl.when(j == HALF)
        def _():
            pltpu.sync_copy(packed_hbm.at[sid, 1], packed_v)

        @pl.when(j < NCHUNK)
        def _():
            @pl.when(j >= 4)
            def _():
                scatter_wait(j - 4, s)
            jj = lax.rem(j, HALF)
            unpack(jj, s)
            gather(s).start()

        @pl.when(j >= 1)
        def _():
            sp = (j - 1) & 3
            gather(sp).wait()
            scatter_start(j - 1, sp)
        return carry
    lax.fori_loop(0, NCHUNK + 1, step, 0)

    def drain(t, carry):
        j = NCHUNK - 4 + t
        scatter_wait(j, j & 3)
        return carry
    lax.fori_loop(0, 4, drain, 0)

    plsc.subcore_barrier()

    def writeout(k, carry):
        u = sid + NS * k

        @pl.when(u < NUNIT)
        def _():
            pltpu.sync_copy(acc_sh.at[pl.ds(u * RU, RU)],
                            out_sum.at[cid, pl.ds(u * RU, RU)])
            pltpu.sync_copy(deg_sh.at[pl.ds(u * RU, RU)],
                            out_deg.at[cid, pl.ds(u * RU, RU)])
        return carry
    lax.fori_loop(0, (NUNIT + NS - 1) // NS, writeout, 0)


_sc_segsum = pl.kernel(
    _sc_body,
    out_type=[jax.ShapeDtypeStruct((NC, N, DH), jnp.float32),
              jax.ShapeDtypeStruct((NC, N, DEGW), jnp.float32)],
    mesh=plsc.VectorSubcoreMesh(core_axis_name="c", subcore_axis_name="s"),
    compiler_params=pltpu.CompilerParams(use_tc_tiling_on_sc=False),
    scratch_types=[
        pltpu.VMEM((HALF, CH), jnp.int32),
        pltpu.VMEM((4, CH), jnp.int32),
        pltpu.VMEM((4, CH), jnp.int32),
        pltpu.VMEM((4, CH, DH), jnp.float32),
        pltpu.VMEM((CH, DEGW), jnp.float32),
        pltpu.VMEM((RU, DEGW), jnp.float32),
        pltpu.VMEM_SHARED((N, DH), jnp.float32),
        pltpu.VMEM_SHARED((N, DH), jnp.float32),
        pltpu.VMEM_SHARED((N, DEGW), jnp.float32),
        pltpu.SemaphoreType.DMA((4,)),
        pltpu.SemaphoreType.DMA((4,)),
        pltpu.SemaphoreType.DMA((4,)),
    ],
)


def _tc_body(ps_ref, pd_ref, feat_ref, w_ref, b_ref, g_ref, be_ref, out_ref):
    summed = jnp.concatenate([ps_ref[0], ps_ref[1]], axis=1)
    deg = (pd_ref[0] + pd_ref[1])[:, 0:1]
    h = summed / jnp.maximum(deg, 1.0)
    z = lax.dot_general(h, w_ref[...],
                        dimension_numbers=(((1,), (1,)), ((), ())),
                        preferred_element_type=jnp.float32)
    z = z + b_ref[...]
    mean = jnp.mean(z, axis=0, keepdims=True)
    c = z - mean
    var = jnp.mean(c * c, axis=0, keepdims=True)
    zn = c / jnp.sqrt(var + EPS) * g_ref[...] + be_ref[...]
    out_ref[...] = feat_ref[...] + jnp.maximum(zn, 0.0)


def kernel(feature, edge_index, W, b, gamma, beta):
    feat_halves = jnp.stack([feature[:, :DH], feature[:, DH:]])
    packed = (edge_index[0] | (edge_index[1] << SHIFT)).reshape(NS, 2, HALF, CH)
    ps, pd = _sc_segsum(feat_halves, packed)
    return pl.pallas_call(
        _tc_body,
        out_shape=jax.ShapeDtypeStruct((N, D), jnp.float32),
    )(ps, pd, feature, W, b.reshape(1, D), gamma.reshape(1, D),
      beta.reshape(1, D))


# speedup vs baseline: 1.2333x; 1.1493x over previous
import jax
import jax.numpy as jnp
from jax import lax
from jax.experimental import pallas as pl
from jax.experimental.pallas import tpu as pltpu
from jax.experimental.pallas import tpu_sc as plsc

N = 10000
D = 128
E = 320000
EPS = 1e-5

NC = 2
NS = 16
DH = D // NC
DW = DH + 8
ESUB = E // NS
CH = 80
NCHUNK = ESUB // CH
HALF = NCHUNK // 2
RU = 80
NUNIT = N // RU
SHIFT = 14


def _sc_body(feat_hbm, packed_hbm, out_sum,
             packed_v, src_c, dst_c, rows_v,
             table_sh, acc_sh, gsem, ssem):
    cid = lax.axis_index("c")
    sid = lax.axis_index("s")

    pltpu.sync_copy(packed_hbm.at[sid, 0], packed_v)

    zeros16 = jnp.zeros((16,), jnp.float32)

    def fill(r, carry):
        for q in range(DW // 8 // 2):
            rows_v[0, r, pl.ds(q * 16, 16)] = zeros16
        rows_v[0, r, pl.ds(DW - 16, 16)] = zeros16
        return carry
    lax.fori_loop(0, RU, fill, 0)

    def init_unit(k, carry):
        u = sid + NS * k

        @pl.when(u < NUNIT)
        def _():
            pltpu.sync_copy(rows_v.at[0], acc_sh.at[pl.ds(u * RU, RU)])
            pltpu.sync_copy(feat_hbm.at[cid, pl.ds(u * RU, RU)],
                            table_sh.at[pl.ds(u * RU, RU)])
        return carry
    lax.fori_loop(0, (NUNIT + NS - 1) // NS, init_unit, 0)

    plsc.subcore_barrier()

    mask = jnp.full((16,), (1 << SHIFT) - 1, jnp.int32)

    def unpack(j, slot):
        for q in range(CH // 16):
            p = packed_v[j, pl.ds(q * 16, 16)]
            src_c[slot, pl.ds(q * 16, 16)] = p & mask
            dst_c[slot, pl.ds(q * 16, 16)] = jax.lax.shift_right_logical(p, SHIFT)

    def gather(slot):
        return pltpu.make_async_copy(table_sh.at[src_c.at[slot]],
                                     rows_v.at[slot], gsem.at[slot])

    def acc_desc(slot):
        return pltpu.make_async_copy(rows_v.at[slot],
                                     acc_sh.at[dst_c.at[slot]], ssem.at[slot])

    def step(j, carry):
        s = j & 3

        @pl.when(j == HALF)
        def _():
            pltpu.sync_copy(packed_hbm.at[sid, 1], packed_v)

        @pl.when(j < NCHUNK)
        def _():
            @pl.when(j >= 4)
            def _():
                acc_desc(s).wait()
            jj = lax.rem(j, HALF)
            unpack(jj, s)
            gather(s).start()

        @pl.when(j >= 1)
        def _():
            sp = (j - 1) & 3
            gather(sp).wait()
            pltpu.async_copy(rows_v.at[sp], acc_sh.at[dst_c.at[sp]],
                             ssem.at[sp], add=True)
        return carry
    lax.fori_loop(0, NCHUNK + 1, step, 0)

    def drain(t, carry):
        acc_desc((NCHUNK - 4 + t) & 3).wait()
        return carry
    lax.fori_loop(0, 4, drain, 0)

    plsc.subcore_barrier()

    def writeout(k, carry):
        u = sid + NS * k

        @pl.when(u < NUNIT)
        def _():
            pltpu.sync_copy(acc_sh.at[pl.ds(u * RU, RU)],
                            out_sum.at[cid, pl.ds(u * RU, RU)])
        return carry
    lax.fori_loop(0, (NUNIT + NS - 1) // NS, writeout, 0)


_sc_segsum = pl.kernel(
    _sc_body,
    out_type=jax.ShapeDtypeStruct((NC, N, DW), jnp.float32),
    mesh=plsc.VectorSubcoreMesh(core_axis_name="c", subcore_axis_name="s"),
    compiler_params=pltpu.CompilerParams(use_tc_tiling_on_sc=False),
    scratch_types=[
        pltpu.VMEM((HALF, CH), jnp.int32),
        pltpu.VMEM((4, CH), jnp.int32),
        pltpu.VMEM((4, CH), jnp.int32),
        pltpu.VMEM((4, CH, DW), jnp.float32),
        pltpu.VMEM_SHARED((N, DW), jnp.float32),
        pltpu.VMEM_SHARED((N, DW), jnp.float32),
        pltpu.SemaphoreType.DMA((4,)),
        pltpu.SemaphoreType.DMA((4,)),
    ],
)


def _tc_body(ps_ref, feat_ref, w_ref, b_ref, g_ref, be_ref, out_ref):
    p0 = ps_ref[0]
    p1 = ps_ref[1]
    summed = jnp.concatenate([p0[:, :DH], p1[:, :DH]], axis=1)
    deg = p0[:, DH:DH + 1]
    h = summed / jnp.maximum(deg, 1.0)
    z = lax.dot_general(h, w_ref[...],
                        dimension_numbers=(((1,), (1,)), ((), ())),
                        preferred_element_type=jnp.float32)
    z = z + b_ref[...]
    mean = jnp.mean(z, axis=0, keepdims=True)
    c = z - mean
    var = jnp.mean(c * c, axis=0, keepdims=True)
    zn = c / jnp.sqrt(var + EPS) * g_ref[...] + be_ref[...]
    out_ref[...] = feat_ref[...] + jnp.maximum(zn, 0.0)


def kernel(feature, edge_index, W, b, gamma, beta):
    ones8 = jnp.ones((N, DW - DH), feature.dtype)
    feat_aug = jnp.stack([
        jnp.concatenate([feature[:, :DH], ones8], axis=1),
        jnp.concatenate([feature[:, DH:], ones8], axis=1),
    ])
    packed = (edge_index[0] | (edge_index[1] << SHIFT)).reshape(NS, 2, HALF, CH)
    ps = _sc_segsum(feat_aug, packed)
    return pl.pallas_call(
        _tc_body,
        out_shape=jax.ShapeDtypeStruct((N, D), jnp.float32),
    )(ps, feature, W, b.reshape(1, D), gamma.reshape(1, D),
      beta.reshape(1, D))


# speedup vs baseline: 1.2372x; 1.0032x over previous
import jax
import jax.numpy as jnp
from jax import lax
from jax.experimental import pallas as pl
from jax.experimental.pallas import tpu as pltpu
from jax.experimental.pallas import tpu_sc as plsc

N = 10000
D = 128
E = 320000
EPS = 1e-5

NC = 2
NS = 16
DH = D // NC
DW = DH + 8
ESUB = E // NS
CH = 80
NCHUNK = ESUB // CH
HALF = NCHUNK // 2
RU = 80
NUNIT = N // RU
SHIFT = 14


def _sc_body(feat_hbm, packed_hbm, out_sum,
             packed_v, src_c, dst_c, rows_v,
             table_sh, acc_sh, gsem, ssem):
    cid = lax.axis_index("c")
    sid = lax.axis_index("s")

    pltpu.sync_copy(packed_hbm.at[sid, 0], packed_v)

    zeros16 = jnp.zeros((16,), jnp.float32)

    def fill(r, carry):
        for q in range(DW // 8 // 2):
            rows_v[0, r, pl.ds(q * 16, 16)] = zeros16
        rows_v[0, r, pl.ds(DW - 16, 16)] = zeros16
        return carry
    lax.fori_loop(0, RU, fill, 0)

    def init_unit(k, carry):
        u = sid + NS * k

        @pl.when(u < NUNIT)
        def _():
            pltpu.sync_copy(rows_v.at[0], acc_sh.at[pl.ds(u * RU, RU)])
            pltpu.sync_copy(feat_hbm.at[cid, pl.ds(u * RU, RU)],
                            table_sh.at[pl.ds(u * RU, RU)])
        return carry
    lax.fori_loop(0, (NUNIT + NS - 1) // NS, init_unit, 0)

    plsc.subcore_barrier()

    mask = jnp.full((16,), (1 << SHIFT) - 1, jnp.int32)

    def unpack(j, slot):
        for q in range(CH // 16):
            p = packed_v[j, pl.ds(q * 16, 16)]
            src_c[slot, pl.ds(q * 16, 16)] = p & mask
            dst_c[slot, pl.ds(q * 16, 16)] = jax.lax.shift_right_logical(p, SHIFT)

    def gather(slot):
        return pltpu.make_async_copy(table_sh.at[src_c.at[slot]],
                                     rows_v.at[slot], gsem.at[slot])

    def acc_desc(slot):
        return pltpu.make_async_copy(rows_v.at[slot],
                                     acc_sh.at[dst_c.at[slot]], ssem.at[slot])

    def step(j, carry):
        s = j & 3

        @pl.when(j == HALF)
        def _():
            pltpu.sync_copy(packed_hbm.at[sid, 1], packed_v)

        @pl.when(j < NCHUNK)
        def _():
            @pl.when(j >= 4)
            def _():
                acc_desc(s).wait()
            jj = lax.rem(j, HALF)
            unpack(jj, s)
            gather(s).start()

        @pl.when(j >= 2)
        def _():
            sp = (j - 2) & 3
            gather(sp).wait()
            pltpu.async_copy(rows_v.at[sp], acc_sh.at[dst_c.at[sp]],
                             ssem.at[sp], add=True)
        return carry
    lax.fori_loop(0, NCHUNK + 2, step, 0)

    def drain(t, carry):
        acc_desc((NCHUNK - 4 + t) & 3).wait()
        return carry
    lax.fori_loop(0, 4, drain, 0)

    plsc.subcore_barrier()

    def writeout(k, carry):
        u = sid + NS * k

        @pl.when(u < NUNIT)
        def _():
            pltpu.sync_copy(acc_sh.at[pl.ds(u * RU, RU)],
                            out_sum.at[cid, pl.ds(u * RU, RU)])
        return carry
    lax.fori_loop(0, (NUNIT + NS - 1) // NS, writeout, 0)


_sc_segsum = pl.kernel(
    _sc_body,
    out_type=jax.ShapeDtypeStruct((NC, N, DW), jnp.float32),
    mesh=plsc.VectorSubcoreMesh(core_axis_name="c", subcore_axis_name="s"),
    compiler_params=pltpu.CompilerParams(use_tc_tiling_on_sc=False),
    scratch_types=[
        pltpu.VMEM((HALF, CH), jnp.int32),
        pltpu.VMEM((4, CH), jnp.int32),
        pltpu.VMEM((4, CH), jnp.int32),
        pltpu.VMEM((4, CH, DW), jnp.float32),
        pltpu.VMEM_SHARED((N, DW), jnp.float32),
        pltpu.VMEM_SHARED((N, DW), jnp.float32),
        pltpu.SemaphoreType.DMA((4,)),
        pltpu.SemaphoreType.DMA((4,)),
    ],
)


def _tc_body(ps_ref, feat_ref, w_ref, b_ref, g_ref, be_ref, out_ref):
    p0 = ps_ref[0]
    p1 = ps_ref[1]
    summed = jnp.concatenate([p0[:, :DH], p1[:, :DH]], axis=1)
    deg = p0[:, DH:DH + 1]
    h = summed / jnp.maximum(deg, 1.0)
    z = lax.dot_general(h, w_ref[...],
                        dimension_numbers=(((1,), (1,)), ((), ())),
                        preferred_element_type=jnp.float32)
    z = z + b_ref[...]
    mean = jnp.mean(z, axis=0, keepdims=True)
    c = z - mean
    var = jnp.mean(c * c, axis=0, keepdims=True)
    zn = c / jnp.sqrt(var + EPS) * g_ref[...] + be_ref[...]
    out_ref[...] = feat_ref[...] + jnp.maximum(zn, 0.0)


def kernel(feature, edge_index, W, b, gamma, beta):
    ones8 = jnp.ones((N, DW - DH), feature.dtype)
    feat_aug = jnp.stack([
        jnp.concatenate([feature[:, :DH], ones8], axis=1),
        jnp.concatenate([feature[:, DH:], ones8], axis=1),
    ])
    packed = (edge_index[0] | (edge_index[1] << SHIFT)).reshape(NS, 2, HALF, CH)
    ps = _sc_segsum(feat_aug, packed)
    return pl.pallas_call(
        _tc_body,
        out_shape=jax.ShapeDtypeStruct((N, D), jnp.float32),
    )(ps, feature, W, b.reshape(1, D), gamma.reshape(1, D),
      beta.reshape(1, D))


# speedup vs baseline: 1.3439x; 1.0862x over previous
import jax
import jax.numpy as jnp
from jax import lax
from jax.experimental import pallas as pl
from jax.experimental.pallas import tpu as pltpu
from jax.experimental.pallas import tpu_sc as plsc

N = 10000
D = 128
E = 320000
EPS = 1e-5

NC = 2
NS = 16
DH = D // NC
DW = DH + 8
ESUB = E // NS
CH = 80
NCHUNK = ESUB // CH
HALF = NCHUNK // 2
RU = 80
NUNIT = N // RU
SHIFT = 14


def _sc_body(feat_hbm, packed_hbm, out_sum,
             packed_v, src_c, dst_c, rows_v,
             table_sh, acc_sh, gsem, ssem):
    cid = lax.axis_index("c")
    sid = lax.axis_index("s")

    pltpu.sync_copy(packed_hbm.at[sid, 0], packed_v)

    zeros16 = jnp.zeros((16,), jnp.float32)
    ones16 = jnp.ones((16,), jnp.float32)

    def fill(r, carry):
        for q in range(DW // 8 // 2):
            rows_v[0, r, pl.ds(q * 16, 16)] = zeros16
        rows_v[0, r, pl.ds(DW - 16, 16)] = zeros16
        rows_v[1, r, pl.ds(0, 16)] = ones16
        return carry
    lax.fori_loop(0, RU, fill, 0)

    def init_unit(k, carry):
        u = sid + NS * k

        @pl.when(u < NUNIT)
        def _():
            pltpu.sync_copy(rows_v.at[0], acc_sh.at[pl.ds(u * RU, RU)])
            pltpu.sync_copy(feat_hbm.at[pl.ds(u * RU, RU), pl.ds(cid * DH, DH)],
                            table_sh.at[pl.ds(u * RU, RU), pl.ds(0, DH)])
            pltpu.sync_copy(rows_v.at[1, pl.ds(0, RU), pl.ds(0, DW - DH)],
                            table_sh.at[pl.ds(u * RU, RU), pl.ds(DH, DW - DH)])
        return carry
    lax.fori_loop(0, (NUNIT + NS - 1) // NS, init_unit, 0)

    plsc.subcore_barrier()

    mask = jnp.full((16,), (1 << SHIFT) - 1, jnp.int32)

    def unpack(j, slot):
        for q in range(CH // 16):
            p = packed_v[j, pl.ds(q * 16, 16)]
            src_c[slot, pl.ds(q * 16, 16)] = p & mask
            dst_c[slot, pl.ds(q * 16, 16)] = jax.lax.shift_right_logical(p, SHIFT)

    def gather(slot):
        return pltpu.make_async_copy(table_sh.at[src_c.at[slot]],
                                     rows_v.at[slot], gsem.at[slot])

    def acc_desc(slot):
        return pltpu.make_async_copy(rows_v.at[slot],
                                     acc_sh.at[dst_c.at[slot]], ssem.at[slot])

    def step(j, carry):
        s = j & 3

        @pl.when(j == HALF)
        def _():
            pltpu.sync_copy(packed_hbm.at[sid, 1], packed_v)

        @pl.when(j < NCHUNK)
        def _():
            @pl.when(j >= 4)
            def _():
                acc_desc(s).wait()
            jj = lax.rem(j, HALF)
            unpack(jj, s)
            gather(s).start()

        @pl.when(j >= 2)
        def _():
            sp = (j - 2) & 3
            gather(sp).wait()
            pltpu.async_copy(rows_v.at[sp], acc_sh.at[dst_c.at[sp]],
                             ssem.at[sp], add=True)
        return carry
    lax.fori_loop(0, NCHUNK + 2, step, 0)

    def drain(t, carry):
        acc_desc((NCHUNK - 4 + t) & 3).wait()
        return carry
    lax.fori_loop(0, 4, drain, 0)

    plsc.subcore_barrier()

    def writeout(k, carry):
        u = sid + NS * k

        @pl.when(u < NUNIT)
        def _():
            pltpu.sync_copy(acc_sh.at[pl.ds(u * RU, RU)],
                            out_sum.at[cid, pl.ds(u * RU, RU)])
        return carry
    lax.fori_loop(0, (NUNIT + NS - 1) // NS, writeout, 0)


_sc_segsum = pl.kernel(
    _sc_body,
    out_type=jax.ShapeDtypeStruct((NC, N, DW), jnp.float32),
    mesh=plsc.VectorSubcoreMesh(core_axis_name="c", subcore_axis_name="s"),
    compiler_params=pltpu.CompilerParams(use_tc_tiling_on_sc=False),
    scratch_types=[
        pltpu.VMEM((HALF, CH), jnp.int32),
        pltpu.VMEM((4, CH), jnp.int32),
        pltpu.VMEM((4, CH), jnp.int32),
        pltpu.VMEM((4, CH, DW), jnp.float32),
        pltpu.VMEM_SHARED((N, DW), jnp.float32),
        pltpu.VMEM_SHARED((N, DW), jnp.float32),
        pltpu.SemaphoreType.DMA((4,)),
        pltpu.SemaphoreType.DMA((4,)),
    ],
)


def _tc_body(ps_ref, feat_ref, w_ref, b_ref, g_ref, be_ref, out_ref):
    p0 = ps_ref[0]
    p1 = ps_ref[1]
    summed = jnp.concatenate([p0[:, :DH], p1[:, :DH]], axis=1)
    deg = p0[:, DH:DH + 1]
    h = summed / jnp.maximum(deg, 1.0)
    z = lax.dot_general(h, w_ref[...],
                        dimension_numbers=(((1,), (1,)), ((), ())),
                        preferred_element_type=jnp.float32)
    z = z + b_ref[...]
    mean = jnp.mean(z, axis=0, keepdims=True)
    c = z - mean
    var = jnp.mean(c * c, axis=0, keepdims=True)
    zn = c / jnp.sqrt(var + EPS) * g_ref[...] + be_ref[...]
    out_ref[...] = feat_ref[...] + jnp.maximum(zn, 0.0)


def kernel(feature, edge_index, W, b, gamma, beta):
    packed = (edge_index[0] | (edge_index[1] << SHIFT)).reshape(NS, 2, HALF, CH)
    ps = _sc_segsum(feature, packed)
    return pl.pallas_call(
        _tc_body,
        out_shape=jax.ShapeDtypeStruct((N, D), jnp.float32),
    )(ps, feature, W, b.reshape(1, D), gamma.reshape(1, D),
      beta.reshape(1, D))


# speedup vs baseline: 1.4138x; 1.0520x over previous
import jax
import jax.numpy as jnp
from jax import lax
from jax.experimental import pallas as pl
from jax.experimental.pallas import tpu as pltpu
from jax.experimental.pallas import tpu_sc as plsc

N = 10000
D = 128
E = 320000
EPS = 1e-5

NC = 2
NS = 16
DH = D // NC
DW = DH + 8
ESUB = E // NS
CH = 80
NCHUNK = ESUB // CH
HALF = NCHUNK // 2
RU = 80
NUNIT = N // RU
SHIFT = 14


def _sc_body(feat_hbm, packed_hbm, out_sum,
             packed_v, src_c, dst_c, rows_v,
             table_sh, acc_sh, gsem, ssem):
    cid = lax.axis_index("c")
    sid = lax.axis_index("s")

    pltpu.sync_copy(packed_hbm.at[sid, 0], packed_v)

    zeros16 = jnp.zeros((16,), jnp.float32)
    ones16 = jnp.ones((16,), jnp.float32)

    def fill(r, carry):
        for q in range(DW // 8 // 2):
            rows_v[0, r, pl.ds(q * 16, 16)] = zeros16
        rows_v[0, r, pl.ds(DW - 16, 16)] = zeros16
        rows_v[1, r, pl.ds(0, 16)] = ones16
        return carry
    lax.fori_loop(0, RU, fill, 0)

    def init_descs(u):
        return (
            pltpu.make_async_copy(rows_v.at[0], acc_sh.at[pl.ds(u * RU, RU)],
                                  ssem.at[0]),
            pltpu.make_async_copy(
                feat_hbm.at[pl.ds(u * RU, RU), pl.ds(cid * DH, DH)],
                table_sh.at[pl.ds(u * RU, RU), pl.ds(0, DH)], ssem.at[1]),
            pltpu.make_async_copy(
                rows_v.at[1, pl.ds(0, RU), pl.ds(0, DW - DH)],
                table_sh.at[pl.ds(u * RU, RU), pl.ds(DH, DW - DH)], ssem.at[2]),
        )

    def init_unit(k, carry):
        u = sid + NS * k

        @pl.when(u < NUNIT)
        def _():
            for d in init_descs(u):
                d.start()
        return carry
    lax.fori_loop(0, (NUNIT + NS - 1) // NS, init_unit, 0)

    def init_drain(k, carry):
        u = sid + NS * k

        @pl.when(u < NUNIT)
        def _():
            for d in init_descs(u):
                d.wait()
        return carry
    lax.fori_loop(0, (NUNIT + NS - 1) // NS, init_drain, 0)

    plsc.subcore_barrier()

    mask = jnp.full((16,), (1 << SHIFT) - 1, jnp.int32)

    def unpack(j, slot):
        for q in range(CH // 16):
            p = packed_v[j, pl.ds(q * 16, 16)]
            src_c[slot, pl.ds(q * 16, 16)] = p & mask
            dst_c[slot, pl.ds(q * 16, 16)] = jax.lax.shift_right_logical(p, SHIFT)

    def gather(slot):
        return pltpu.make_async_copy(table_sh.at[src_c.at[slot]],
                                     rows_v.at[slot], gsem.at[slot])

    def acc_desc(slot):
        return pltpu.make_async_copy(rows_v.at[slot],
                                     acc_sh.at[dst_c.at[slot]], ssem.at[slot])

    def step(j, carry):
        s = j & 3

        @pl.when(j == HALF)
        def _():
            pltpu.sync_copy(packed_hbm.at[sid, 1], packed_v)

        @pl.when(j < NCHUNK)
        def _():
            @pl.when(j >= 4)
            def _():
                acc_desc(s).wait()
            jj = lax.rem(j, HALF)
            unpack(jj, s)
            gather(s).start()

        @pl.when(j >= 2)
        def _():
            sp = (j - 2) & 3
            gather(sp).wait()
            pltpu.async_copy(rows_v.at[sp], acc_sh.at[dst_c.at[sp]],
                             ssem.at[sp], add=True)
        return carry
    lax.fori_loop(0, NCHUNK + 2, step, 0)

    def drain(t, carry):
        acc_desc((NCHUNK - 4 + t) & 3).wait()
        return carry
    lax.fori_loop(0, 4, drain, 0)

    plsc.subcore_barrier()

    def out_desc(u):
        return pltpu.make_async_copy(acc_sh.at[pl.ds(u * RU, RU)],
                                     out_sum.at[cid, pl.ds(u * RU, RU)],
                                     ssem.at[0])

    def writeout(k, carry):
        u = sid + NS * k

        @pl.when(u < NUNIT)
        def _():
            out_desc(u).start()
        return carry
    lax.fori_loop(0, (NUNIT + NS - 1) // NS, writeout, 0)

    def writeout_drain(k, carry):
        u = sid + NS * k

        @pl.when(u < NUNIT)
        def _():
            out_desc(u).wait()
        return carry
    lax.fori_loop(0, (NUNIT + NS - 1) // NS, writeout_drain, 0)


_sc_segsum = pl.kernel(
    _sc_body,
    out_type=jax.ShapeDtypeStruct((NC, N, DW), jnp.float32),
    mesh=plsc.VectorSubcoreMesh(core_axis_name="c", subcore_axis_name="s"),
    compiler_params=pltpu.CompilerParams(use_tc_tiling_on_sc=False),
    scratch_types=[
        pltpu.VMEM((HALF, CH), jnp.int32),
        pltpu.VMEM((4, CH), jnp.int32),
        pltpu.VMEM((4, CH), jnp.int32),
        pltpu.VMEM((4, CH, DW), jnp.float32),
        pltpu.VMEM_SHARED((N, DW), jnp.float32),
        pltpu.VMEM_SHARED((N, DW), jnp.float32),
        pltpu.SemaphoreType.DMA((4,)),
        pltpu.SemaphoreType.DMA((4,)),
    ],
)


def _tc_body(ps_ref, feat_ref, w_ref, b_ref, g_ref, be_ref, out_ref):
    p0 = ps_ref[0]
    p1 = ps_ref[1]
    summed = jnp.concatenate([p0[:, :DH], p1[:, :DH]], axis=1)
    deg = p0[:, DH:DH + 1]
    h = summed / jnp.maximum(deg, 1.0)
    z = lax.dot_general(h, w_ref[...],
                        dimension_numbers=(((1,), (1,)), ((), ())),
                        preferred_element_type=jnp.float32)
    z = z + b_ref[...]
    mean = jnp.mean(z, axis=0, keepdims=True)
    c = z - mean
    var = jnp.mean(c * c, axis=0, keepdims=True)
    zn = c / jnp.sqrt(var + EPS) * g_ref[...] + be_ref[...]
    out_ref[...] = feat_ref[...] + jnp.maximum(zn, 0.0)


def kernel(feature, edge_index, W, b, gamma, beta):
    packed = (edge_index[0] | (edge_index[1] << SHIFT)).reshape(NS, 2, HALF, CH)
    ps = _sc_segsum(feature, packed)
    return pl.pallas_call(
        _tc_body,
        out_shape=jax.ShapeDtypeStruct((N, D), jnp.float32),
    )(ps, feature, W, b.reshape(1, D), gamma.reshape(1, D),
      beta.reshape(1, D))
